# baseline (device time: 295984 ns/iter reference)
import jax
import jax.numpy as jnp
from jax import lax
from jax.experimental import pallas as pl
from jax.experimental.pallas import tpu as pltpu

N_DEV = 4


def kernel(x, k, Wp):
    b, s, c_sh = x.shape
    taps = k.shape[0]
    n_out = Wp.shape[1]

    def body(x_ref, k_ref, w_ref, out_ref, comm_ref, send_sems, recv_sems):
        my = lax.axis_index("i")
        left = (my - 1) % N_DEV
        right = (my + 1) % N_DEV

        xv = x_ref[...]
        kv = k_ref[...]
        acc = xv * jnp.reshape(kv[taps - 1], (1, 1, c_sh))
        for j in range(1, taps):
            t = taps - 1 - j
            shifted = jnp.concatenate(
                [jnp.zeros((b, j, c_sh), jnp.float32), xv[:, : s - j, :]],
                axis=1,
            )
            acc = acc + shifted * jnp.reshape(kv[t], (1, 1, c_sh))
        a = acc * jax.nn.sigmoid(acc)
        partial = lax.dot_general(
            a,
            w_ref[...],
            dimension_numbers=(((2,), (0,)), ((), ())),
            preferred_element_type=jnp.float32,
        )

        out_ref[...] = partial
        comm_ref[0] = partial

        barrier_sem = pltpu.get_barrier_semaphore()
        for nbr in (left, right):
            pl.semaphore_signal(
                barrier_sem,
                inc=1,
                device_id=(nbr,),
                device_id_type=pl.DeviceIdType.MESH,
            )
        pl.semaphore_wait(barrier_sem, 2)

        for h in range(N_DEV - 1):
            send_slot = h % 2
            recv_slot = (h + 1) % 2
            rdma = pltpu.make_async_remote_copy(
                src_ref=comm_ref.at[send_slot],
                dst_ref=comm_ref.at[recv_slot],
                send_sem=send_sems.at[send_slot],
                recv_sem=recv_sems.at[recv_slot],
                device_id=(right,),
                device_id_type=pl.DeviceIdType.MESH,
            )
            rdma.start()
            rdma.wait()
            out_ref[...] = out_ref[...] + comm_ref[recv_slot]

    return pl.pallas_call(
        body,
        out_shape=jax.ShapeDtypeStruct((b, s, n_out), jnp.float32),
        in_specs=[
            pl.BlockSpec(memory_space=pltpu.VMEM),
            pl.BlockSpec(memory_space=pltpu.VMEM),
            pl.BlockSpec(memory_space=pltpu.VMEM),
        ],
        out_specs=pl.BlockSpec(memory_space=pltpu.VMEM),
        scratch_shapes=[
            pltpu.VMEM((2, b, s, n_out), jnp.float32),
            pltpu.SemaphoreType.DMA((2,)),
            pltpu.SemaphoreType.DMA((2,)),
        ],
        compiler_params=pltpu.CompilerParams(collective_id=0),
    )(x, k, Wp)


# device time: 96866 ns/iter; 3.0556x vs baseline; 3.0556x over previous
import jax
import jax.numpy as jnp
from jax import lax
from jax.experimental import pallas as pl
from jax.experimental.pallas import tpu as pltpu

N_DEV = 4


def kernel(x, k, Wp):
    b, s, c_sh = x.shape
    taps = k.shape[0]
    n_out = Wp.shape[1]
    bh = b // 2
    ch = s // N_DEV

    def body(x_ref, k_ref, w_ref, out_ref, comm_r, comm_l, send_r, recv_r,
             send_l, recv_l):
        d = lax.axis_index("i")
        left = (d - 1) % N_DEV
        right = (d + 1) % N_DEV

        xv = x_ref[...]
        kv = k_ref[...]
        acc = xv * jnp.reshape(kv[taps - 1], (1, 1, c_sh))
        for j in range(1, taps):
            t = taps - 1 - j
            shifted = jnp.concatenate(
                [jnp.zeros((b, j, c_sh), jnp.float32), xv[:, : s - j, :]],
                axis=1,
            )
            acc = acc + shifted * jnp.reshape(kv[t], (1, 1, c_sh))
        a = acc * jax.nn.sigmoid(acc)
        out_ref[...] = lax.dot_general(
            a,
            w_ref[...],
            dimension_numbers=(((2,), (0,)), ((), ())),
            preferred_element_type=jnp.float32,
        )

        comm_r[0] = out_ref[0:bh, pl.ds(d * ch, ch), :]
        comm_l[0] = out_ref[bh:b, pl.ds(d * ch, ch), :]

        barrier_sem = pltpu.get_barrier_semaphore()
        for nbr in (left, right):
            pl.semaphore_signal(
                barrier_sem,
                inc=1,
                device_id=(nbr,),
                device_id_type=pl.DeviceIdType.MESH,
            )
        pl.semaphore_wait(barrier_sem, 2)

        for h in range(2 * (N_DEV - 1)):
            ss = h % 2
            rs = (h + 1) % 2
            rdma_r = pltpu.make_async_remote_copy(
                src_ref=comm_r.at[ss],
                dst_ref=comm_r.at[rs],
                send_sem=send_r.at[ss],
                recv_sem=recv_r.at[rs],
                device_id=(right,),
                device_id_type=pl.DeviceIdType.MESH,
            )
            rdma_l = pltpu.make_async_remote_copy(
                src_ref=comm_l.at[ss],
                dst_ref=comm_l.at[rs],
                send_sem=send_l.at[ss],
                recv_sem=recv_l.at[rs],
                device_id=(left,),
                device_id_type=pl.DeviceIdType.MESH,
            )
            rdma_r.start()
            rdma_l.start()
            rdma_r.wait()
            rdma_l.wait()
            if h < N_DEV - 1:
                c_r = (d - h - 1) % N_DEV
                c_l = (d + h + 1) % N_DEV
                comm_r[rs] = comm_r[rs] + out_ref[0:bh, pl.ds(c_r * ch, ch), :]
                comm_l[rs] = comm_l[rs] + out_ref[bh:b, pl.ds(c_l * ch, ch), :]
                if h == N_DEV - 2:
                    out_ref[0:bh, pl.ds(c_r * ch, ch), :] = comm_r[rs]
                    out_ref[bh:b, pl.ds(c_l * ch, ch), :] = comm_l[rs]
            else:
                g = h - (N_DEV - 1)
                c_r = (d - g) % N_DEV
                c_l = (d + g) % N_DEV
                out_ref[0:bh, pl.ds(c_r * ch, ch), :] = comm_r[rs]
                out_ref[bh:b, pl.ds(c_l * ch, ch), :] = comm_l[rs]

    return pl.pallas_call(
        body,
        out_shape=jax.ShapeDtypeStruct((b, s, n_out), jnp.float32),
        in_specs=[
            pl.BlockSpec(memory_space=pltpu.VMEM),
            pl.BlockSpec(memory_space=pltpu.VMEM),
            pl.BlockSpec(memory_space=pltpu.VMEM),
        ],
        out_specs=pl.BlockSpec(memory_space=pltpu.VMEM),
        scratch_shapes=[
            pltpu.VMEM((2, bh, ch, n_out), jnp.float32),
            pltpu.VMEM((2, bh, ch, n_out), jnp.float32),
            pltpu.SemaphoreType.DMA((2,)),
            pltpu.SemaphoreType.DMA((2,)),
            pltpu.SemaphoreType.DMA((2,)),
            pltpu.SemaphoreType.DMA((2,)),
        ],
        compiler_params=pltpu.CompilerParams(collective_id=0),
    )(x, k, Wp)


# device time: 93004 ns/iter; 3.1825x vs baseline; 1.0415x over previous
import jax
import jax.numpy as jnp
from jax import lax
from jax.experimental import pallas as pl
from jax.experimental.pallas import tpu as pltpu

N_DEV = 4


def kernel(x, k, Wp):
    b, s, c_sh = x.shape
    taps = k.shape[0]
    n_out = Wp.shape[1]
    bh = b // 2
    sh = s // 2
    sq = s // 4

    def body(x_ref, k_ref, w_ref, out_ref, r_a1, r_b1, r_a2, r_b2, r_a3,
             r_b3, r_a4, r_b4, send_sems, recv_sems):
        d = lax.axis_index("i")
        p = d + 1 - 2 * jnp.remainder(d, 2)
        q = 3 - d

        half_a = jnp.where(d <= 1, 0, sh)
        send_a1 = sh - half_a
        q_a = d * sq
        send_a2 = 2 * half_a + sq - q_a
        q_a_peer = p * sq
        half_b = jnp.where((d == 0) | (d == 3), 0, sh)
        send_b1 = sh - half_b
        q_b = half_b + jnp.where(d >= 2, sq, 0)
        send_b2 = 2 * half_b + sq - q_b
        q_b_peer = half_b + jnp.where(d <= 1, sq, 0)

        xv = x_ref[...]
        kv = k_ref[...]
        acc = xv * jnp.reshape(kv[taps - 1], (1, 1, c_sh))
        for j in range(1, taps):
            t = taps - 1 - j
            shifted = jnp.concatenate(
                [jnp.zeros((b, j, c_sh), jnp.float32), xv[:, : s - j, :]],
                axis=1,
            )
            acc = acc + shifted * jnp.reshape(kv[t], (1, 1, c_sh))
        a = acc * jax.nn.sigmoid(acc)
        out_ref[...] = lax.dot_general(
            a,
            w_ref[...],
            dimension_numbers=(((2,), (0,)), ((), ())),
            preferred_element_type=jnp.float32,
        )

        barrier_sem = pltpu.get_barrier_semaphore()
        for nbr in (p, q):
            pl.semaphore_signal(
                barrier_sem,
                inc=1,
                device_id=(nbr,),
                device_id_type=pl.DeviceIdType.MESH,
            )
        pl.semaphore_wait(barrier_sem, 2)

        def rcopy(src, dst, sem_idx, target):
            return pltpu.make_async_remote_copy(
                src_ref=src,
                dst_ref=dst,
                send_sem=send_sems.at[sem_idx],
                recv_sem=recv_sems.at[sem_idx],
                device_id=(target,),
                device_id_type=pl.DeviceIdType.MESH,
            )

        a1 = rcopy(out_ref.at[0:bh, pl.ds(send_a1, sh), :], r_a1, 0, q)
        b1 = rcopy(out_ref.at[bh:b, pl.ds(send_b1, sh), :], r_b1, 1, p)
        a1.start()
        b1.start()
        a1.wait()
        b1.wait()
        out_ref[0:bh, pl.ds(half_a, sh), :] = (
            out_ref[0:bh, pl.ds(half_a, sh), :] + r_a1[...]
        )
        out_ref[bh:b, pl.ds(half_b, sh), :] = (
            out_ref[bh:b, pl.ds(half_b, sh), :] + r_b1[...]
        )

        a2 = rcopy(out_ref.at[0:bh, pl.ds(send_a2, sq), :], r_a2, 2, p)
        b2 = rcopy(out_ref.at[bh:b, pl.ds(send_b2, sq), :], r_b2, 3, q)
        a2.start()
        b2.start()
        a2.wait()
        b2.wait()
        out_ref[0:bh, pl.ds(q_a, sq), :] = (
            out_ref[0:bh, pl.ds(q_a, sq), :] + r_a2[...]
        )
        out_ref[bh:b, pl.ds(q_b, sq), :] = (
            out_ref[bh:b, pl.ds(q_b, sq), :] + r_b2[...]
        )

        a3 = rcopy(out_ref.at[0:bh, pl.ds(q_a, sq), :], r_a3, 4, p)
        b3 = rcopy(out_ref.at[bh:b, pl.ds(q_b, sq), :], r_b3, 5, q)
        a3.start()
        b3.start()
        a3.wait()
        b3.wait()
        out_ref[0:bh, pl.ds(q_a_peer, sq), :] = r_a3[...]
        out_ref[bh:b, pl.ds(q_b_peer, sq), :] = r_b3[...]

        a4 = rcopy(out_ref.at[0:bh, pl.ds(half_a, sh), :], r_a4, 6, q)
        b4 = rcopy(out_ref.at[bh:b, pl.ds(half_b, sh), :], r_b4, 7, p)
        a4.start()
        b4.start()
        a4.wait()
        b4.wait()
        out_ref[0:bh, pl.ds(send_a1, sh), :] = r_a4[...]
        out_ref[bh:b, pl.ds(send_b1, sh), :] = r_b4[...]

    return pl.pallas_call(
        body,
        out_shape=jax.ShapeDtypeStruct((b, s, n_out), jnp.float32),
        in_specs=[
            pl.BlockSpec(memory_space=pltpu.VMEM),
            pl.BlockSpec(memory_space=pltpu.VMEM),
            pl.BlockSpec(memory_space=pltpu.VMEM),
        ],
        out_specs=pl.BlockSpec(memory_space=pltpu.VMEM),
        scratch_shapes=[
            pltpu.VMEM((bh, sh, n_out), jnp.float32),
            pltpu.VMEM((bh, sh, n_out), jnp.float32),
            pltpu.VMEM((bh, sq, n_out), jnp.float32),
            pltpu.VMEM((bh, sq, n_out), jnp.float32),
            pltpu.VMEM((bh, sq, n_out), jnp.float32),
            pltpu.VMEM((bh, sq, n_out), jnp.float32),
            pltpu.VMEM((bh, sh, n_out), jnp.float32),
            pltpu.VMEM((bh, sh, n_out), jnp.float32),
            pltpu.SemaphoreType.DMA((8,)),
            pltpu.SemaphoreType.DMA((8,)),
        ],
        compiler_params=pltpu.CompilerParams(collective_id=0),
    )(x, k, Wp)


# device time: 59411 ns/iter; 4.9820x vs baseline; 1.5654x over previous
import jax
import jax.numpy as jnp
from jax import lax
from jax.experimental import pallas as pl
from jax.experimental.pallas import tpu as pltpu

N_DEV = 4


def kernel(x, k, Wp):
    b, s, c_sh = x.shape
    taps = k.shape[0]
    n_out = Wp.shape[1]
    bh = b // 2
    sh = s // 2
    sq = s // 4

    def body(x_ref, k_ref, w_ref, out_ref, a_ref, p16, r_a1, r_b1, r_a2,
             r_b2, send_sems, recv_sems):
        d = lax.axis_index("i")
        p = d + 1 - 2 * jnp.remainder(d, 2)
        q = 3 - d

        half_a = jnp.where(d <= 1, 0, sh)
        send_a1 = sh - half_a
        q_a = d * sq
        send_a2 = 2 * half_a + sq - q_a
        half_b = jnp.where((d == 0) | (d == 3), 0, sh)
        send_b1 = sh - half_b
        q_b = half_b + jnp.where(d >= 2, sq, 0)
        send_b2 = 2 * half_b + sq - q_b

        xv = x_ref[...]
        kv = k_ref[...]
        acc = xv * jnp.reshape(kv[taps - 1], (1, 1, c_sh))
        for j in range(1, taps):
            t = taps - 1 - j
            shifted = jnp.concatenate(
                [jnp.zeros((b, j, c_sh), jnp.float32), xv[:, : s - j, :]],
                axis=1,
            )
            acc = acc + shifted * jnp.reshape(kv[t], (1, 1, c_sh))
        a_ref[...] = acc * jax.nn.sigmoid(acc)

        def mm_region(r0, r1, st, ln):
            p16[r0:r1, pl.ds(st, ln), :] = lax.dot_general(
                a_ref[r0:r1, pl.ds(st, ln), :],
                w_ref[...],
                dimension_numbers=(((2,), (0,)), ((), ())),
                preferred_element_type=jnp.float32,
            ).astype(jnp.bfloat16)

        mm_region(0, bh, send_a1, sh)
        mm_region(bh, b, send_b1, sh)

        barrier_sem = pltpu.get_barrier_semaphore()
        for nbr in (p, q):
            pl.semaphore_signal(
                barrier_sem,
                inc=1,
                device_id=(nbr,),
                device_id_type=pl.DeviceIdType.MESH,
            )
        pl.semaphore_wait(barrier_sem, 2)

        def rcopy(src, dst, sem_idx, target):
            return pltpu.make_async_remote_copy(
                src_ref=src,
                dst_ref=dst,
                send_sem=send_sems.at[sem_idx],
                recv_sem=recv_sems.at[sem_idx],
                device_id=(target,),
                device_id_type=pl.DeviceIdType.MESH,
            )

        a1 = rcopy(p16.at[0:bh, pl.ds(send_a1, sh), :], r_a1, 0, q)
        b1 = rcopy(p16.at[bh:b, pl.ds(send_b1, sh), :], r_b1, 1, p)
        a1.start()
        b1.start()

        mm_region(0, bh, half_a, sh)
        mm_region(bh, b, half_b, sh)

        a1.wait()
        b1.wait()
        p16[0:bh, pl.ds(send_a2, sq), :] = (
            p16[0:bh, pl.ds(send_a2, sq), :]
            + r_a1[:, pl.ds(send_a2 - half_a, sq), :]
        )
        p16[bh:b, pl.ds(send_b2, sq), :] = (
            p16[bh:b, pl.ds(send_b2, sq), :]
            + r_b1[:, pl.ds(send_b2 - half_b, sq), :]
        )

        a2 = rcopy(p16.at[0:bh, pl.ds(send_a2, sq), :], r_a2, 2, p)
        b2 = rcopy(p16.at[bh:b, pl.ds(send_b2, sq), :], r_b2, 3, q)
        a2.start()
        b2.start()

        p16[0:bh, pl.ds(q_a, sq), :] = (
            p16[0:bh, pl.ds(q_a, sq), :]
            + r_a1[:, pl.ds(q_a - half_a, sq), :]
        )
        p16[bh:b, pl.ds(q_b, sq), :] = (
            p16[bh:b, pl.ds(q_b, sq), :]
            + r_b1[:, pl.ds(q_b - half_b, sq), :]
        )

        a2.wait()
        b2.wait()
        p16[0:bh, pl.ds(q_a, sq), :] = (
            p16[0:bh, pl.ds(q_a, sq), :] + r_a2[...]
        )
        p16[bh:b, pl.ds(q_b, sq), :] = (
            p16[bh:b, pl.ds(q_b, sq), :] + r_b2[...]
        )

        a3 = rcopy(
            p16.at[0:bh, pl.ds(q_a, sq), :],
            p16.at[0:bh, pl.ds(q_a, sq), :],
            4,
            p,
        )
        b3 = rcopy(
            p16.at[bh:b, pl.ds(q_b, sq), :],
            p16.at[bh:b, pl.ds(q_b, sq), :],
            5,
            q,
        )
        a3.start()
        b3.start()
        a3.wait()
        b3.wait()

        a4 = rcopy(
            p16.at[0:bh, pl.ds(half_a, sh), :],
            p16.at[0:bh, pl.ds(half_a, sh), :],
            6,
            q,
        )
        b4 = rcopy(
            p16.at[bh:b, pl.ds(half_b, sh), :],
            p16.at[bh:b, pl.ds(half_b, sh), :],
            7,
            p,
        )
        a4.start()
        b4.start()

        out_ref[0:bh, pl.ds(half_a, sh), :] = (
            p16[0:bh, pl.ds(half_a, sh), :].astype(jnp.float32)
        )
        out_ref[bh:b, pl.ds(half_b, sh), :] = (
            p16[bh:b, pl.ds(half_b, sh), :].astype(jnp.float32)
        )

        a4.wait()
        b4.wait()
        out_ref[0:bh, pl.ds(send_a1, sh), :] = (
            p16[0:bh, pl.ds(send_a1, sh), :].astype(jnp.float32)
        )
        out_ref[bh:b, pl.ds(send_b1, sh), :] = (
            p16[bh:b, pl.ds(send_b1, sh), :].astype(jnp.float32)
        )

    return pl.pallas_call(
        body,
        out_shape=jax.ShapeDtypeStruct((b, s, n_out), jnp.float32),
        in_specs=[
            pl.BlockSpec(memory_space=pltpu.VMEM),
            pl.BlockSpec(memory_space=pltpu.VMEM),
            pl.BlockSpec(memory_space=pltpu.VMEM),
        ],
        out_specs=pl.BlockSpec(memory_space=pltpu.VMEM),
        scratch_shapes=[
            pltpu.VMEM((b, s, c_sh), jnp.float32),
            pltpu.VMEM((b, s, n_out), jnp.bfloat16),
            pltpu.VMEM((bh, sh, n_out), jnp.bfloat16),
            pltpu.VMEM((bh, sh, n_out), jnp.bfloat16),
            pltpu.VMEM((bh, sq, n_out), jnp.bfloat16),
            pltpu.VMEM((bh, sq, n_out), jnp.bfloat16),
            pltpu.SemaphoreType.DMA((8,)),
            pltpu.SemaphoreType.DMA((8,)),
        ],
        compiler_params=pltpu.CompilerParams(collective_id=0),
    )(x, k, Wp)
